# baseline (speedup 1.0000x reference)
import functools

import jax
import jax.numpy as jnp
from jax import lax
from jax.experimental import pallas as pl
from jax.experimental.pallas import tpu as pltpu
from jax.experimental.pallas import tpu_sc as plsc

V = 100000
RANK = 32
K1 = 32
B = 128
NDIM = 3
NS = 16
BPW = B // NS
RPB = K1 * RANK // 128
C1 = 1000000.0 / 128.0
HALF_LOG_2PI = 0.9189385332046727
LOSS_CONST = 1000000.0 * HALF_LOG_2PI


def _vlog(x):
    xi = lax.bitcast_convert_type(x, jnp.int32)
    e = lax.shift_right_logical(xi, 23) - 127
    mi = lax.bitwise_or(lax.bitwise_and(xi, jnp.int32(0x007FFFFF)),
                        jnp.int32(0x3F800000))
    m = lax.bitcast_convert_type(mi, jnp.float32)
    big = m > jnp.float32(1.4142135623730951)
    m = jnp.where(big, m * jnp.float32(0.5), m)
    e = e + jnp.where(big, jnp.int32(1), jnp.int32(0))
    z = (m - jnp.float32(1.0)) / (m + jnp.float32(1.0))
    z2 = z * z
    p = jnp.float32(1.0 / 9.0)
    p = p * z2 + jnp.float32(1.0 / 7.0)
    p = p * z2 + jnp.float32(1.0 / 5.0)
    p = p * z2 + jnp.float32(1.0 / 3.0)
    p = p * z2 + jnp.float32(1.0)
    return jnp.float32(2.0) * z * p + e.astype(jnp.float32) * jnp.float32(0.6931471805599453)


def _flatten_body(t0, t1, t2, t3, t4, t5, out_ref, sem):
    tabs = (t0, t1, t2, t3, t4, t5)
    copies = []
    for t in range(2 * NDIM):
        for c in range(RANK):
            cp = pltpu.make_async_copy(
                tabs[t].at[c], out_ref.at[pl.ds((t * RANK + c) * V, V)], sem)
            cp.start()
            copies.append(cp)
    for cp in copies:
        cp.wait()


_flatten = pl.pallas_call(
    _flatten_body,
    out_shape=jax.ShapeDtypeStruct((2 * NDIM * RANK * V,), jnp.float32),
    in_specs=[pl.BlockSpec(memory_space=pl.ANY)] * (2 * NDIM),
    out_specs=pl.BlockSpec(memory_space=pl.ANY),
    scratch_shapes=[pltpu.SemaphoreType.DMA],
)


def _body(tabs_h, ys_h, eps_h, i0, i1, i2, out_h,
          idx_v, gidx_v, m_v, l_v, eps_v, ys_v, red_v, all_v, shared_v, sem):
    cid = lax.axis_index("c")
    sid = lax.axis_index("s")

    @pl.when(cid == 0)
    def _core0():
        base = sid * BPW
        idxs = (i0, i1, i2)
        for d in range(NDIM):
            pltpu.sync_copy(idxs[d].at[pl.ds(base, BPW)], idx_v.at[pl.ds(d * BPW, BPW)])
        pltpu.sync_copy(ys_h.at[pl.ds(base, BPW)], ys_v.at[pl.ds(0, BPW)])
        iv_lo = idx_v[0:16]
        iv_hi = idx_v[16:32]
        lo16 = lax.iota(jnp.int32, 16)
        cstride_lo = lo16 * V
        cstride_hi = cstride_lo + 16 * V
        coff = jnp.full((16,), NDIM * RANK * V, jnp.int32)
        for d in range(NDIM):
            for b in range(BPW):
                row = iv_hi[b] if d == 2 else iv_lo[d * BPW + b]
                g = d * BPW + b
                bvec = jnp.full((16,), d * (RANK * V), jnp.int32) + row
                j, col = g // 4, (g % 4) * RANK
                gidx_v[j, col:col + 16] = bvec + cstride_lo
                gidx_v[j, col + 16:col + 32] = bvec + cstride_hi
                gidx_v[j + 6, col:col + 16] = bvec + cstride_lo + coff
                gidx_v[j + 6, col + 16:col + 32] = bvec + cstride_hi + coff
        handles = []
        for j in range(2 * NDIM):
            handles.append(pltpu.async_copy(
                tabs_h.at[gidx_v.at[j]], m_v.at[pl.ds(j * 128, 128)], sem))
            handles.append(pltpu.async_copy(
                tabs_h.at[gidx_v.at[j + 6]], l_v.at[pl.ds(j * 128, 128)], sem))
        for d in range(NDIM):
            handles.append(pltpu.async_copy(
                eps_h.at[pl.ds(d * (B * RPB) + base * RPB, BPW * RPB)],
                eps_v.at[d], sem))
        for h in handles:
            h.wait()

        yall = ys_v[0:16]
        acc_be = jnp.zeros((16,), jnp.float32)
        acc_kl = jnp.zeros((16,), jnp.float32)
        for b in range(BPW):
            regs = []
            for d in range(NDIM):
                g = d * BPW + b
                ma = m_v[g * RANK:g * RANK + 16]
                mb = m_v[g * RANK + 16:g * RANK + 32]
                la = l_v[g * RANK:g * RANK + 16]
                lb = l_v[g * RANK + 16:g * RANK + 32]
                lla = la * la
                llb = lb * lb
                regs.append((ma, mb, lla, llb))
                for mm, ll in ((ma, lla), (mb, llb)):
                    vr = ll * ll
                    acc_kl = acc_kl + (vr + mm * mm - jnp.float32(1.0) - _vlog(vr))

            yv = jnp.full((16,), yall[b], jnp.float32)

            def kbody(k, accv, b=b, regs=regs, yv=yv):
                fa = None
                fb = None
                rr = b * RPB + lax.shift_right_logical(k, 2)
                cc = lax.bitwise_and(k, 3) * RANK
                for d in range(NDIM):
                    ma, mb, lla, llb = regs[d]
                    ea = eps_v[d, rr, pl.ds(cc, 16)]
                    eb = eps_v[d, rr, pl.ds(cc + 16, 16)]
                    ga = ma + ea * lla
                    gb = mb + eb * llb
                    fa = ga if fa is None else fa * ga
                    fb = gb if fb is None else fb * gb
                dv = yv - jnp.full((16,), jnp.sum(fa + fb), jnp.float32)
                return accv + dv * dv

            accv = lax.fori_loop(0, K1, kbody, jnp.zeros((16,), jnp.float32))
            acc_be = acc_be + accv * jnp.float32(1.0 / 16.0)

        pv = acc_be * jnp.float32(C1 / 64.0) + acc_kl * jnp.float32(1.0 / 256.0)
        red_v[0:16] = pv
        pltpu.sync_copy(red_v.at[pl.ds(0, 16)], shared_v.at[sid, pl.ds(0, 16)])
        plsc.subcore_barrier()

        @pl.when(sid == 0)
        def _final():
            pltpu.sync_copy(shared_v, all_v)
            tot = jnp.zeros((16,), jnp.float32)
            for i in range(NS):
                tot = tot + all_v[i, 0:16]
            tot = jnp.cumsum(tot)
            tot = lax.rev(tot, (0,))
            tot = tot + jnp.float32(LOSS_CONST)
            red_v[0:16] = tot
            pltpu.sync_copy(red_v.at[pl.ds(0, 1)], out_h)


_mesh = plsc.VectorSubcoreMesh(core_axis_name="c", subcore_axis_name="s")

_sc_loss = functools.partial(
    pl.kernel,
    out_type=jax.ShapeDtypeStruct((1,), jnp.float32),
    mesh=_mesh,
    compiler_params=pltpu.CompilerParams(needs_layout_passes=False,
                                         use_tc_tiling_on_sc=True),
    scratch_types=[
        pltpu.VMEM((128,), jnp.int32),
        pltpu.VMEM((4 * NDIM, 128), jnp.int32),
        pltpu.VMEM((NDIM * BPW * RANK,), jnp.float32),
        pltpu.VMEM((NDIM * BPW * RANK,), jnp.float32),
        pltpu.VMEM((NDIM, B * RPB // NS, 128), jnp.float32),
        pltpu.VMEM((128,), jnp.float32),
        pltpu.VMEM((128,), jnp.float32),
        pltpu.VMEM((NS, 128), jnp.float32),
        pltpu.VMEM_SHARED((NS, 128), jnp.float32),
        pltpu.SemaphoreType.DMA,
    ],
)(_body)


def kernel(means0, means1, means2, chols0, chols1, chols2, ys, eps, entries):
    idx = entries.astype(jnp.int32).T
    eps_rows = eps.reshape(NDIM * B * K1 * RANK // 128, 128)
    tabs = jnp.concatenate([means0.T, means1.T, means2.T,
                            chols0.T, chols1.T, chols2.T], axis=0).reshape(-1)
    return _sc_loss(tabs, ys, eps_rows, idx[0], idx[1], idx[2])


# speedup vs baseline: 1.4267x; 1.4267x over previous
import functools

import jax
import jax.numpy as jnp
from jax import lax
from jax.experimental import pallas as pl
from jax.experimental.pallas import tpu as pltpu
from jax.experimental.pallas import tpu_sc as plsc

V = 100000
RANK = 32
K1 = 32
B = 128
NDIM = 3
NS = 16
BPW = B // NS
RPB = K1 * RANK // 128
C1 = 1000000.0 / 128.0
HALF_LOG_2PI = 0.9189385332046727
LOSS_CONST = 1000000.0 * HALF_LOG_2PI


def _vlog(x):
    xi = lax.bitcast_convert_type(x, jnp.int32)
    e = lax.shift_right_logical(xi, 23) - 127
    mi = lax.bitwise_or(lax.bitwise_and(xi, jnp.int32(0x007FFFFF)),
                        jnp.int32(0x3F800000))
    m = lax.bitcast_convert_type(mi, jnp.float32)
    big = m > jnp.float32(1.4142135623730951)
    m = jnp.where(big, m * jnp.float32(0.5), m)
    e = e + jnp.where(big, jnp.int32(1), jnp.int32(0))
    z = (m - jnp.float32(1.0)) / (m + jnp.float32(1.0))
    z2 = z * z
    p = jnp.float32(1.0 / 9.0)
    p = p * z2 + jnp.float32(1.0 / 7.0)
    p = p * z2 + jnp.float32(1.0 / 5.0)
    p = p * z2 + jnp.float32(1.0 / 3.0)
    p = p * z2 + jnp.float32(1.0)
    return jnp.float32(2.0) * z * p + e.astype(jnp.float32) * jnp.float32(0.6931471805599453)


VP = 100096


VP = 100096
_QR = 8
_NQ = RANK // _QR


def _flatten_body(t0, t1, t2, t3, t4, t5, out_ref):
    s = pl.program_id(0)
    ins = (t0, t1, t2, t3, t4, t5)
    for i in range(2 * NDIM):
        @pl.when(s // _NQ == i)
        def _(i=i):
            out_ref[0, :, 0:V] = ins[i][...]


def _in_spec(i):
    return pl.BlockSpec(
        (_QR, V), lambda s, i=i: (jnp.clip(s - _NQ * i, 0, _NQ - 1), 0))


_flatten = pl.pallas_call(
    _flatten_body,
    grid=(2 * NDIM * _NQ,),
    in_specs=[_in_spec(i) for i in range(2 * NDIM)],
    out_specs=pl.BlockSpec((1, _QR, VP), lambda s: (s, 0, 0)),
    out_shape=jax.ShapeDtypeStruct((2 * NDIM * _NQ, _QR, VP), jnp.float32),
)


def _body(tabs_h, ys_h, eps_h, i0, i1, i2, out_h,
          idx_v, gidx_v, m_v, l_v, eps_v, ys_v, red_v, all_v, shared_v, sem):
    cid = lax.axis_index("c")
    sid = lax.axis_index("s")

    @pl.when(cid == 0)
    def _core0():
        base = sid * BPW
        idxs = (i0, i1, i2)
        for d in range(NDIM):
            pltpu.sync_copy(idxs[d].at[pl.ds(base, BPW)], idx_v.at[pl.ds(d * BPW, BPW)])
        pltpu.sync_copy(ys_h.at[pl.ds(base, BPW)], ys_v.at[pl.ds(0, BPW)])
        iv_lo = idx_v[0:16]
        iv_hi = idx_v[16:32]
        lo16 = lax.iota(jnp.int32, 16)
        cstride_lo = lo16 * VP
        cstride_hi = cstride_lo + 16 * VP
        coff = jnp.full((16,), NDIM * RANK * VP, jnp.int32)
        for d in range(NDIM):
            for b in range(BPW):
                row = iv_hi[b] if d == 2 else iv_lo[d * BPW + b]
                g = d * BPW + b
                bvec = jnp.full((16,), d * (RANK * VP), jnp.int32) + row
                j, col = g // 4, (g % 4) * RANK
                gidx_v[j, col:col + 16] = bvec + cstride_lo
                gidx_v[j, col + 16:col + 32] = bvec + cstride_hi
                gidx_v[j + 6, col:col + 16] = bvec + cstride_lo + coff
                gidx_v[j + 6, col + 16:col + 32] = bvec + cstride_hi + coff
        handles = []
        for j in range(2 * NDIM):
            handles.append(pltpu.async_copy(
                tabs_h.at[gidx_v.at[j]], m_v.at[pl.ds(j * 128, 128)], sem))
            handles.append(pltpu.async_copy(
                tabs_h.at[gidx_v.at[j + 6]], l_v.at[pl.ds(j * 128, 128)], sem))
        for d in range(NDIM):
            handles.append(pltpu.async_copy(
                eps_h.at[pl.ds(d * (B * RPB) + base * RPB, BPW * RPB)],
                eps_v.at[d], sem))
        for h in handles:
            h.wait()

        yall = ys_v[0:16]
        acc_be = jnp.zeros((16,), jnp.float32)
        acc_kl = jnp.zeros((16,), jnp.float32)
        for b in range(BPW):
            regs = []
            for d in range(NDIM):
                g = d * BPW + b
                ma = m_v[g * RANK:g * RANK + 16]
                mb = m_v[g * RANK + 16:g * RANK + 32]
                la = l_v[g * RANK:g * RANK + 16]
                lb = l_v[g * RANK + 16:g * RANK + 32]
                lla = la * la
                llb = lb * lb
                regs.append((ma, mb, lla, llb))
                for mm, ll in ((ma, lla), (mb, llb)):
                    vr = ll * ll
                    acc_kl = acc_kl + (vr + mm * mm - jnp.float32(1.0) - _vlog(vr))

            yv = jnp.full((16,), yall[b], jnp.float32)

            def kbody(k, accv, b=b, regs=regs, yv=yv):
                fa = None
                fb = None
                rr = b * RPB + lax.shift_right_logical(k, 2)
                cc = lax.bitwise_and(k, 3) * RANK
                for d in range(NDIM):
                    ma, mb, lla, llb = regs[d]
                    ea = eps_v[d, rr, pl.ds(cc, 16)]
                    eb = eps_v[d, rr, pl.ds(cc + 16, 16)]
                    ga = ma + ea * lla
                    gb = mb + eb * llb
                    fa = ga if fa is None else fa * ga
                    fb = gb if fb is None else fb * gb
                dv = yv - jnp.full((16,), jnp.sum(fa + fb), jnp.float32)
                return accv + dv * dv

            accv = lax.fori_loop(0, K1, kbody, jnp.zeros((16,), jnp.float32))
            acc_be = acc_be + accv * jnp.float32(1.0 / 16.0)

        pv = acc_be * jnp.float32(C1 / 64.0) + acc_kl * jnp.float32(1.0 / 256.0)
        red_v[0:16] = pv
        pltpu.sync_copy(red_v.at[pl.ds(0, 16)], shared_v.at[sid, pl.ds(0, 16)])
        plsc.subcore_barrier()

        @pl.when(sid == 0)
        def _final():
            pltpu.sync_copy(shared_v, all_v)
            tot = jnp.zeros((16,), jnp.float32)
            for i in range(NS):
                tot = tot + all_v[i, 0:16]
            tot = jnp.cumsum(tot)
            tot = lax.rev(tot, (0,))
            tot = tot + jnp.float32(LOSS_CONST)
            red_v[0:16] = tot
            pltpu.sync_copy(red_v.at[pl.ds(0, 1)], out_h)


_mesh = plsc.VectorSubcoreMesh(core_axis_name="c", subcore_axis_name="s")

_sc_loss = functools.partial(
    pl.kernel,
    out_type=jax.ShapeDtypeStruct((1,), jnp.float32),
    mesh=_mesh,
    compiler_params=pltpu.CompilerParams(needs_layout_passes=False,
                                         use_tc_tiling_on_sc=True),
    scratch_types=[
        pltpu.VMEM((128,), jnp.int32),
        pltpu.VMEM((4 * NDIM, 128), jnp.int32),
        pltpu.VMEM((NDIM * BPW * RANK,), jnp.float32),
        pltpu.VMEM((NDIM * BPW * RANK,), jnp.float32),
        pltpu.VMEM((NDIM, B * RPB // NS, 128), jnp.float32),
        pltpu.VMEM((128,), jnp.float32),
        pltpu.VMEM((128,), jnp.float32),
        pltpu.VMEM((NS, 128), jnp.float32),
        pltpu.VMEM_SHARED((NS, 128), jnp.float32),
        pltpu.SemaphoreType.DMA,
    ],
)(_body)


def kernel(means0, means1, means2, chols0, chols1, chols2, ys, eps, entries):
    idx = entries.astype(jnp.int32).T
    eps_rows = eps.reshape(NDIM * B * K1 * RANK // 128, 128)
    tabs = _flatten(means0.T, means1.T, means2.T,
                    chols0.T, chols1.T, chols2.T).reshape(-1)
    return _sc_loss(tabs, ys, eps_rows, idx[0], idx[1], idx[2])


# speedup vs baseline: 1.4282x; 1.0010x over previous
import functools

import jax
import jax.numpy as jnp
from jax import lax
from jax.experimental import pallas as pl
from jax.experimental.pallas import tpu as pltpu
from jax.experimental.pallas import tpu_sc as plsc

V = 100000
RANK = 32
K1 = 32
B = 128
NDIM = 3
NS = 16
BPW = B // NS
RPB = K1 * RANK // 128
C1 = 1000000.0 / 128.0
HALF_LOG_2PI = 0.9189385332046727
LOSS_CONST = 1000000.0 * HALF_LOG_2PI


def _vlog(x):
    xi = lax.bitcast_convert_type(x, jnp.int32)
    e = lax.shift_right_logical(xi, 23) - 127
    mi = lax.bitwise_or(lax.bitwise_and(xi, jnp.int32(0x007FFFFF)),
                        jnp.int32(0x3F800000))
    m = lax.bitcast_convert_type(mi, jnp.float32)
    big = m > jnp.float32(1.4142135623730951)
    m = jnp.where(big, m * jnp.float32(0.5), m)
    e = e + jnp.where(big, jnp.int32(1), jnp.int32(0))
    z = (m - jnp.float32(1.0)) / (m + jnp.float32(1.0))
    z2 = z * z
    p = jnp.float32(1.0 / 9.0)
    p = p * z2 + jnp.float32(1.0 / 7.0)
    p = p * z2 + jnp.float32(1.0 / 5.0)
    p = p * z2 + jnp.float32(1.0 / 3.0)
    p = p * z2 + jnp.float32(1.0)
    return jnp.float32(2.0) * z * p + e.astype(jnp.float32) * jnp.float32(0.6931471805599453)


VP = 100096
_QR = 8
_NQ = RANK // _QR


def _flatten_body(t0, t1, t2, t3, t4, t5, out_ref):
    s = pl.program_id(0)
    ins = (t0, t1, t2, t3, t4, t5)
    for i in range(2 * NDIM):
        @pl.when(s // _NQ == i)
        def _(i=i):
            out_ref[0, :, 0:V] = ins[i][...]


def _in_spec(i):
    return pl.BlockSpec(
        (_QR, V), lambda s, i=i: (jnp.clip(s - _NQ * i, 0, _NQ - 1), 0))


_flatten = pl.pallas_call(
    _flatten_body,
    grid=(2 * NDIM * _NQ,),
    in_specs=[_in_spec(i) for i in range(2 * NDIM)],
    out_specs=pl.BlockSpec((1, _QR, VP), lambda s: (s, 0, 0)),
    out_shape=jax.ShapeDtypeStruct((2 * NDIM * _NQ, _QR, VP), jnp.float32),
)


def _body(tabs_h, ys_h, eps_h, i0, i1, i2, out_h,
          idx_v, gidx_v, m_v, l_v, eps_v, ys_v, red_v, all_v, shared_v, sem):
    cid = lax.axis_index("c")
    sid = lax.axis_index("s")

    @pl.when(cid == 0)
    def _core0():
        base = sid * BPW
        idxs = (i0, i1, i2)
        for d in range(NDIM):
            pltpu.sync_copy(idxs[d].at[pl.ds(base, BPW)], idx_v.at[pl.ds(d * BPW, BPW)])
        pltpu.sync_copy(ys_h.at[pl.ds(base, BPW)], ys_v.at[pl.ds(0, BPW)])
        iv_lo = idx_v[0:16]
        iv_hi = idx_v[16:32]
        lo16 = lax.iota(jnp.int32, 16)
        cstride_lo = lo16 * VP
        cstride_hi = cstride_lo + 16 * VP
        coff = jnp.full((16,), NDIM * RANK * VP, jnp.int32)
        for d in range(NDIM):
            for b in range(BPW):
                row = iv_hi[b] if d == 2 else iv_lo[d * BPW + b]
                g = d * BPW + b
                bvec = jnp.full((16,), d * (RANK * VP), jnp.int32) + row
                j, col = g // 4, (g % 4) * RANK
                gidx_v[j, col:col + 16] = bvec + cstride_lo
                gidx_v[j, col + 16:col + 32] = bvec + cstride_hi
                gidx_v[j + 6, col:col + 16] = bvec + cstride_lo + coff
                gidx_v[j + 6, col + 16:col + 32] = bvec + cstride_hi + coff
        handles = []
        for j in range(2 * NDIM):
            handles.append(pltpu.async_copy(
                tabs_h.at[gidx_v.at[j]], m_v.at[pl.ds(j * 128, 128)], sem))
            handles.append(pltpu.async_copy(
                tabs_h.at[gidx_v.at[j + 6]], l_v.at[pl.ds(j * 128, 128)], sem))
        for d in range(NDIM):
            handles.append(pltpu.async_copy(
                eps_h.at[pl.ds(d * (B * RPB) + base * RPB, BPW * RPB)],
                eps_v.at[d], sem))
        for h in handles:
            h.wait()

        yall = ys_v[0:16]
        acc_be = jnp.zeros((16,), jnp.float32)
        acc_kl = jnp.zeros((16,), jnp.float32)
        for b in range(BPW):
            regs = []
            for d in range(NDIM):
                g = d * BPW + b
                ma = m_v[g * RANK:g * RANK + 16]
                mb = m_v[g * RANK + 16:g * RANK + 32]
                la = l_v[g * RANK:g * RANK + 16]
                lb = l_v[g * RANK + 16:g * RANK + 32]
                lla = la * la
                llb = lb * lb
                regs.append((ma, mb, lla, llb))
                for mm, ll in ((ma, lla), (mb, llb)):
                    vr = ll * ll
                    acc_kl = acc_kl + (vr + mm * mm - jnp.float32(1.0) - _vlog(vr))

            yv = jnp.full((16,), yall[b], jnp.float32)

            def kbody(k, accv, b=b, regs=regs, yv=yv):
                fa = None
                fb = None
                rr = b * RPB + lax.shift_right_logical(k, 2)
                cc = lax.bitwise_and(k, 3) * RANK
                for d in range(NDIM):
                    ma, mb, lla, llb = regs[d]
                    ea = eps_v[d, rr, pl.ds(cc, 16)]
                    eb = eps_v[d, rr, pl.ds(cc + 16, 16)]
                    ga = ma + ea * lla
                    gb = mb + eb * llb
                    fa = ga if fa is None else fa * ga
                    fb = gb if fb is None else fb * gb
                dv = yv - jnp.full((16,), jnp.sum(fa + fb), jnp.float32)
                return accv + dv * dv

            accv = lax.fori_loop(0, K1, kbody, jnp.zeros((16,), jnp.float32))
            acc_be = acc_be + accv * jnp.float32(1.0 / 16.0)

        pv = acc_be * jnp.float32(C1 / 64.0) + acc_kl * jnp.float32(1.0 / 256.0)
        red_v[0:16] = pv
        pltpu.sync_copy(red_v.at[pl.ds(0, 16)], shared_v.at[sid, pl.ds(0, 16)])
        plsc.subcore_barrier()

        @pl.when(sid == 0)
        def _final():
            pltpu.sync_copy(shared_v, all_v)
            tot = jnp.zeros((16,), jnp.float32)
            for i in range(NS):
                tot = tot + all_v[i, 0:16]
            tot = jnp.cumsum(tot)
            tot = lax.rev(tot, (0,))
            tot = tot + jnp.float32(LOSS_CONST)
            red_v[0:16] = tot
            pltpu.sync_copy(red_v.at[pl.ds(0, 1)], out_h)


_mesh = plsc.VectorSubcoreMesh(core_axis_name="c", subcore_axis_name="s")

_sc_loss = functools.partial(
    pl.kernel,
    out_type=jax.ShapeDtypeStruct((1,), jnp.float32),
    mesh=_mesh,
    compiler_params=pltpu.CompilerParams(needs_layout_passes=False,
                                         use_tc_tiling_on_sc=True),
    scratch_types=[
        pltpu.VMEM((128,), jnp.int32),
        pltpu.VMEM((4 * NDIM, 128), jnp.int32),
        pltpu.VMEM((NDIM * BPW * RANK,), jnp.float32),
        pltpu.VMEM((NDIM * BPW * RANK,), jnp.float32),
        pltpu.VMEM((NDIM, B * RPB // NS, 128), jnp.float32),
        pltpu.VMEM((128,), jnp.float32),
        pltpu.VMEM((128,), jnp.float32),
        pltpu.VMEM((NS, 128), jnp.float32),
        pltpu.VMEM_SHARED((NS, 128), jnp.float32),
        pltpu.SemaphoreType.DMA,
    ],
)(_body)


def kernel(means0, means1, means2, chols0, chols1, chols2, ys, eps, entries):
    idx = entries.astype(jnp.int32).T
    eps_rows = eps.reshape(NDIM * B * K1 * RANK // 128, 128)
    tabs = _flatten(means0.T, means1.T, means2.T,
                    chols0.T, chols1.T, chols2.T).reshape(-1)
    return _sc_loss(tabs, ys, eps_rows, idx[0], idx[1], idx[2])
